# baseline (speedup 1.0000x reference)
import functools

import jax
import jax.numpy as jnp
from jax.experimental import pallas as pl
from jax.experimental.pallas import tpu as pltpu

N_SUB = 256
SUBS = 8
M_CHUNK = 512


def _emd_kernel(p_ref, t_ref, o_ref, *, m_total):
    t3 = t_ref[0]
    ta = jnp.concatenate(
        [t3 * (-2.0), jnp.zeros((5, m_total), jnp.float32)], axis=0
    )
    t2 = jnp.sum(t3 * t3, axis=0, keepdims=True)

    for s in range(SUBS):
        p = p_ref[0, s * N_SUB:(s + 1) * N_SUB, :]
        acc = None
        for j in range(m_total // M_CHUNK):
            mm = jax.lax.dot_general(
                p, ta[:, j * M_CHUNK:(j + 1) * M_CHUNK],
                (((1,), (0,)), ((), ())),
                preferred_element_type=jnp.float32,
            )
            z = mm + t2[:, j * M_CHUNK:(j + 1) * M_CHUNK]
            for l in range(M_CHUNK // 128):
                tile = z[:, l * 128:(l + 1) * 128]
                acc = tile if acc is None else jnp.minimum(acc, tile)

        zmin = jnp.min(acc, axis=1, keepdims=True)
        p2 = jnp.sum(p * p, axis=1, keepdims=True)
        d2 = jnp.maximum(zmin + p2, 0.0)
        o_ref[0, s * N_SUB:(s + 1) * N_SUB, :] = jnp.sqrt(d2)


def kernel(pred, target):
    B, N, _ = pred.shape
    M = target.shape[1]
    n_blk = N_SUB * SUBS

    zeros = jnp.zeros((B, N, 5), jnp.float32)
    pred_aug = jnp.concatenate([pred, zeros], axis=-1)
    target_t = jnp.transpose(target, (0, 2, 1))

    grid = (B, N // n_blk)
    min_d = pl.pallas_call(
        functools.partial(_emd_kernel, m_total=M),
        grid=grid,
        in_specs=[
            pl.BlockSpec((1, n_blk, 8), lambda b, n: (b, n, 0)),
            pl.BlockSpec((1, 3, M), lambda b, n: (b, 0, 0)),
        ],
        out_specs=pl.BlockSpec((1, n_blk, 1), lambda b, n: (b, n, 0)),
        out_shape=jax.ShapeDtypeStruct((B, N, 1), jnp.float32),
        compiler_params=pltpu.CompilerParams(
            dimension_semantics=("parallel", "arbitrary"),
        ),
    )(pred_aug, target_t)

    return jnp.mean(min_d[..., 0])


# speedup vs baseline: 1.0210x; 1.0210x over previous
import functools

import jax
import jax.numpy as jnp
from jax.experimental import pallas as pl
from jax.experimental.pallas import tpu as pltpu

N_SUB = 256
SUBS = 16
M_CHUNK = 512


def _emd_kernel(p_ref, t_ref, o_ref, *, m_total):
    t3 = t_ref[0]
    ta = jnp.concatenate(
        [t3 * (-2.0), jnp.zeros((5, m_total), jnp.float32)], axis=0
    )
    t2 = jnp.sum(t3 * t3, axis=0, keepdims=True)

    for s in range(SUBS):
        p = p_ref[0, s * N_SUB:(s + 1) * N_SUB, :]
        acc = None
        for j in range(m_total // M_CHUNK):
            mm = jax.lax.dot_general(
                p, ta[:, j * M_CHUNK:(j + 1) * M_CHUNK],
                (((1,), (0,)), ((), ())),
                preferred_element_type=jnp.float32,
            )
            z = mm + t2[:, j * M_CHUNK:(j + 1) * M_CHUNK]
            for l in range(M_CHUNK // 128):
                tile = z[:, l * 128:(l + 1) * 128]
                acc = tile if acc is None else jnp.minimum(acc, tile)

        zmin = jnp.min(acc, axis=1, keepdims=True)
        p2 = jnp.sum(p * p, axis=1, keepdims=True)
        d2 = jnp.maximum(zmin + p2, 0.0)
        o_ref[0, s * N_SUB:(s + 1) * N_SUB, :] = jnp.sqrt(d2)


def kernel(pred, target):
    B, N, _ = pred.shape
    M = target.shape[1]
    n_blk = N_SUB * SUBS

    zeros = jnp.zeros((B, N, 5), jnp.float32)
    pred_aug = jnp.concatenate([pred, zeros], axis=-1)
    target_t = jnp.transpose(target, (0, 2, 1))

    grid = (B, N // n_blk)
    min_d = pl.pallas_call(
        functools.partial(_emd_kernel, m_total=M),
        grid=grid,
        in_specs=[
            pl.BlockSpec((1, n_blk, 8), lambda b, n: (b, n, 0)),
            pl.BlockSpec((1, 3, M), lambda b, n: (b, 0, 0)),
        ],
        out_specs=pl.BlockSpec((1, n_blk, 1), lambda b, n: (b, n, 0)),
        out_shape=jax.ShapeDtypeStruct((B, N, 1), jnp.float32),
        compiler_params=pltpu.CompilerParams(
            dimension_semantics=("parallel", "arbitrary"),
        ),
    )(pred_aug, target_t)

    return jnp.mean(min_d[..., 0])


# speedup vs baseline: 1.0213x; 1.0003x over previous
import functools

import jax
import jax.numpy as jnp
from jax.experimental import pallas as pl
from jax.experimental.pallas import tpu as pltpu

N_SUB = 256
SUBS = 16
M_CHUNK = 512
VPU_TILES = 0


def _emd_kernel(p_ref, t_ref, o_ref, *, m_total):
    t3 = t_ref[0]
    ta = jnp.concatenate(
        [t3 * (-2.0), jnp.zeros((5, m_total), jnp.float32)], axis=0
    )
    t2 = jnp.sum(t3 * t3, axis=0, keepdims=True)

    m_mxu = m_total - VPU_TILES * 128

    for s in range(SUBS):
        p = p_ref[0, s * N_SUB:(s + 1) * N_SUB, :]

        acc = None
        for j in range(m_mxu // M_CHUNK):
            mm = jax.lax.dot_general(
                p, ta[:, j * M_CHUNK:(j + 1) * M_CHUNK],
                (((1,), (0,)), ((), ())),
                preferred_element_type=jnp.float32,
            )
            z = mm + t2[:, j * M_CHUNK:(j + 1) * M_CHUNK]
            for l in range(M_CHUNK // 128):
                tile = z[:, l * 128:(l + 1) * 128]
                acc = tile if acc is None else jnp.minimum(acc, tile)

        zmin = jnp.min(acc, axis=1, keepdims=True)
        p2 = jnp.sum(p * p, axis=1, keepdims=True)
        dmin = zmin + p2

        if VPU_TILES:
            px = jnp.broadcast_to(p[:, 0:1], (N_SUB, 128))
            py = jnp.broadcast_to(p[:, 1:2], (N_SUB, 128))
            pz = jnp.broadcast_to(p[:, 2:3], (N_SUB, 128))
            accb = None
            for l in range(VPU_TILES):
                off = m_mxu + l * 128
                dx = px - t3[0:1, off:off + 128]
                dy = py - t3[1:2, off:off + 128]
                dz = pz - t3[2:3, off:off + 128]
                zb = dx * dx + dy * dy + dz * dz
                accb = zb if accb is None else jnp.minimum(accb, zb)
            bmin = jnp.min(accb, axis=1, keepdims=True)
            dmin = jnp.minimum(dmin, bmin)

        d2 = jnp.maximum(dmin, 0.0)
        o_ref[0, s * N_SUB:(s + 1) * N_SUB, :] = jnp.sqrt(d2)


def kernel(pred, target):
    B, N, _ = pred.shape
    M = target.shape[1]
    n_blk = N_SUB * SUBS

    zeros = jnp.zeros((B, N, 5), jnp.float32)
    pred_aug = jnp.concatenate([pred, zeros], axis=-1)
    target_t = jnp.transpose(target, (0, 2, 1))

    grid = (B, N // n_blk)
    min_d = pl.pallas_call(
        functools.partial(_emd_kernel, m_total=M),
        grid=grid,
        in_specs=[
            pl.BlockSpec((1, n_blk, 8), lambda b, n: (b, n, 0)),
            pl.BlockSpec((1, 3, M), lambda b, n: (b, 0, 0)),
        ],
        out_specs=pl.BlockSpec((1, n_blk, 1), lambda b, n: (b, n, 0)),
        out_shape=jax.ShapeDtypeStruct((B, N, 1), jnp.float32),
        compiler_params=pltpu.CompilerParams(
            dimension_semantics=("parallel", "arbitrary"),
        ),
    )(pred_aug, target_t)

    return jnp.mean(min_d[..., 0])


# speedup vs baseline: 1.1406x; 1.1169x over previous
import functools

import jax
import jax.numpy as jnp
from jax.experimental import pallas as pl
from jax.experimental.pallas import tpu as pltpu

N_SUB = 256
SUBS = 16
M_CHUNK = 512
VPU_TILES = 0


def _emd_kernel(p_ref, t_ref, o_ref, *, m_total):
    t3 = t_ref[0]
    ta = t3 * (-2.0)
    t2 = jnp.sum(t3 * t3, axis=0, keepdims=True)

    m_mxu = m_total - VPU_TILES * 128

    for s in range(SUBS):
        p = p_ref[0, s * N_SUB:(s + 1) * N_SUB, :]

        acc = None
        for j in range(m_mxu // M_CHUNK):
            mm = jax.lax.dot_general(
                p, ta[:, j * M_CHUNK:(j + 1) * M_CHUNK],
                (((1,), (0,)), ((), ())),
                preferred_element_type=jnp.float32,
            )
            z = mm + t2[:, j * M_CHUNK:(j + 1) * M_CHUNK]
            for l in range(M_CHUNK // 128):
                tile = z[:, l * 128:(l + 1) * 128]
                acc = tile if acc is None else jnp.minimum(acc, tile)

        zmin = jnp.min(acc, axis=1, keepdims=True)
        p2 = jnp.sum(p * p, axis=1, keepdims=True)
        dmin = zmin + p2

        if VPU_TILES:
            px = jnp.broadcast_to(p[:, 0:1], (N_SUB, 128))
            py = jnp.broadcast_to(p[:, 1:2], (N_SUB, 128))
            pz = jnp.broadcast_to(p[:, 2:3], (N_SUB, 128))
            accb = None
            for l in range(VPU_TILES):
                off = m_mxu + l * 128
                dx = px - t3[0:1, off:off + 128]
                dy = py - t3[1:2, off:off + 128]
                dz = pz - t3[2:3, off:off + 128]
                zb = dx * dx + dy * dy + dz * dz
                accb = zb if accb is None else jnp.minimum(accb, zb)
            bmin = jnp.min(accb, axis=1, keepdims=True)
            dmin = jnp.minimum(dmin, bmin)

        d2 = jnp.maximum(dmin, 0.0)
        o_ref[0, s * N_SUB:(s + 1) * N_SUB, :] = jnp.sqrt(d2)


def kernel(pred, target):
    B, N, _ = pred.shape
    M = target.shape[1]
    n_blk = N_SUB * SUBS

    target_t = jnp.transpose(target, (0, 2, 1))

    grid = (B, N // n_blk)
    min_d = pl.pallas_call(
        functools.partial(_emd_kernel, m_total=M),
        grid=grid,
        in_specs=[
            pl.BlockSpec((1, n_blk, 3), lambda b, n: (b, n, 0)),
            pl.BlockSpec((1, 3, M), lambda b, n: (b, 0, 0)),
        ],
        out_specs=pl.BlockSpec((1, n_blk, 1), lambda b, n: (b, n, 0)),
        out_shape=jax.ShapeDtypeStruct((B, N, 1), jnp.float32),
        compiler_params=pltpu.CompilerParams(
            dimension_semantics=("parallel", "arbitrary"),
        ),
    )(pred, target_t)

    return jnp.mean(min_d[..., 0])


# speedup vs baseline: 1.3345x; 1.1699x over previous
import functools

import jax
import jax.numpy as jnp
from jax.experimental import pallas as pl
from jax.experimental.pallas import tpu as pltpu

N_SUB = 256
SUBS = 16
M_CHUNK = 512


def _emd_kernel(p_ref, t_ref, o_ref, *, m_total):
    t3 = t_ref[0]
    ones_row = jnp.ones((1, m_total), jnp.float32)
    ta = jnp.concatenate([t3 * (-2.0), ones_row], axis=0)
    t2 = jnp.sum(t3 * t3, axis=0, keepdims=True)

    pt = p_ref[0]

    total = jnp.zeros((1, 1), jnp.float32)
    for s in range(SUBS):
        pts = pt[:, s * N_SUB:(s + 1) * N_SUB]
        p2row = jnp.sum(pts * pts, axis=0, keepdims=True)
        pa = jnp.concatenate([pts, p2row], axis=0)

        acc = None
        for j in range(m_total // M_CHUNK):
            mm = jax.lax.dot_general(
                pa, ta[:, j * M_CHUNK:(j + 1) * M_CHUNK],
                (((0,), (0,)), ((), ())),
                preferred_element_type=jnp.float32,
            )
            z = mm + t2[:, j * M_CHUNK:(j + 1) * M_CHUNK]
            for l in range(M_CHUNK // 128):
                tile = z[:, l * 128:(l + 1) * 128]
                acc = tile if acc is None else jnp.minimum(acc, tile)

        zmin = jnp.min(acc, axis=1, keepdims=True)
        d = jnp.sqrt(jnp.maximum(zmin, 0.0))
        total = total + jnp.sum(d, axis=0, keepdims=True)

    o_ref[0] = total


def kernel(pred, target):
    B, N, _ = pred.shape
    M = target.shape[1]

    pred_t = jnp.transpose(pred, (0, 2, 1))
    target_t = jnp.transpose(target, (0, 2, 1))

    grid = (B,)
    sums = pl.pallas_call(
        functools.partial(_emd_kernel, m_total=M),
        grid=grid,
        in_specs=[
            pl.BlockSpec((1, 3, N), lambda b: (b, 0, 0)),
            pl.BlockSpec((1, 3, M), lambda b: (b, 0, 0)),
        ],
        out_specs=pl.BlockSpec((1, 1, 1), lambda b: (b, 0, 0)),
        out_shape=jax.ShapeDtypeStruct((B, 1, 1), jnp.float32),
        compiler_params=pltpu.CompilerParams(
            dimension_semantics=("arbitrary",),
        ),
    )(pred_t, target_t)

    return jnp.sum(sums) / (B * N)


# speedup vs baseline: 1.3454x; 1.0082x over previous
import functools

import jax
import jax.numpy as jnp
from jax.experimental import pallas as pl
from jax.experimental.pallas import tpu as pltpu

N_SUB = 256
SUBS = 16
M_CHUNK = 512


def _emd_kernel(p_ref, t_ref, o_ref, *, m_total, n_batches, n_rows):
    t3 = t_ref[0]
    ones_row = jnp.ones((1, m_total), jnp.float32)
    ta = jnp.concatenate([t3 * (-2.0), ones_row], axis=0)
    t2 = jnp.sum(t3 * t3, axis=0, keepdims=True)

    pt = p_ref[0]

    total = jnp.zeros((1, 1), jnp.float32)
    for s in range(SUBS):
        pts = pt[:, s * N_SUB:(s + 1) * N_SUB]
        p2row = jnp.sum(pts * pts, axis=0, keepdims=True)
        pa = jnp.concatenate([pts, p2row], axis=0)

        acc = None
        for j in range(m_total // M_CHUNK):
            mm = jax.lax.dot_general(
                pa, ta[:, j * M_CHUNK:(j + 1) * M_CHUNK],
                (((0,), (0,)), ((), ())),
                preferred_element_type=jnp.float32,
            )
            z = mm + t2[:, j * M_CHUNK:(j + 1) * M_CHUNK]
            for l in range(M_CHUNK // 128):
                tile = z[:, l * 128:(l + 1) * 128]
                acc = tile if acc is None else jnp.minimum(acc, tile)

        zmin = jnp.min(acc, axis=1, keepdims=True)
        d = jnp.sqrt(jnp.maximum(zmin, 0.0))
        total = total + jnp.sum(d, axis=0, keepdims=True)

    b = pl.program_id(0)

    @pl.when(b == 0)
    def _():
        o_ref[0] = jnp.zeros((1, 1), jnp.float32)

    o_ref[0] = o_ref[0] + total * (1.0 / (n_batches * n_rows))


def kernel(pred, target):
    B, N, _ = pred.shape
    M = target.shape[1]

    pred_t = jnp.transpose(pred, (0, 2, 1))
    target_t = jnp.transpose(target, (0, 2, 1))

    grid = (B,)
    mean = pl.pallas_call(
        functools.partial(_emd_kernel, m_total=M, n_batches=B, n_rows=N),
        grid=grid,
        in_specs=[
            pl.BlockSpec((1, 3, N), lambda b: (b, 0, 0)),
            pl.BlockSpec((1, 3, M), lambda b: (b, 0, 0)),
        ],
        out_specs=pl.BlockSpec((1, 1, 1), lambda b: (0, 0, 0)),
        out_shape=jax.ShapeDtypeStruct((1, 1, 1), jnp.float32),
        compiler_params=pltpu.CompilerParams(
            dimension_semantics=("arbitrary",),
        ),
    )(pred_t, target_t)

    return mean[0, 0, 0]
